# Initial kernel scaffold; baseline (speedup 1.0000x reference)
import jax
import jax.numpy as jnp
from jax.experimental import pallas as pl


def kernel(x1, x2, edge_index1, edge_weight1, edge_index2, edge_weight2, W_enc1, W_enc2, w_omega, u_omega, W_dec1, W_dec2):
    raise NotImplementedError("write your pallas kernel here")


# speedup vs baseline: 1.5308x; 1.5308x over previous
import functools

import jax
import jax.numpy as jnp
from jax import lax
from jax.experimental import pallas as pl
from jax.experimental.pallas import tpu as pltpu
from jax.experimental.pallas import tpu_sc as plsc

N = 10000
E = 320000
D = 128

NC = 2
NS = 16
EPT = E // NS
C = 80
NCHUNK = EPT // C
RPT = 624
REM = N - NS * RPT


H = D // 2


@functools.partial(
    pl.kernel,
    out_type=[jax.ShapeDtypeStruct((2 * N, H), jnp.float32),
              jax.ShapeDtypeStruct((2 * N, H), jnp.float32)],
    mesh=plsc.VectorSubcoreMesh(core_axis_name="c", subcore_axis_name="s"),
    compiler_params=pltpu.CompilerParams(use_tc_tiling_on_sc=False),
    scratch_types=[
        pltpu.VMEM((C,), jnp.int32),
        pltpu.VMEM((C,), jnp.int32),
        pltpu.VMEM((C,), jnp.float32),
        pltpu.VMEM((C, H), jnp.float32),
        pltpu.VMEM((RPT, H), jnp.float32),
        pltpu.VMEM_SHARED((N, H), jnp.float32),
        pltpu.SemaphoreType.DMA,
    ],
)
def _spmm_pair(xw_lo, xw_hi, src_hbm, dst_hbm, ew_hbm, out_lo, out_hi,
               sidx, didx, wv, rows, obuf, acc, sem):
    c = lax.axis_index("c")
    s = lax.axis_index("s")
    zero16 = jnp.zeros((16,), jnp.float32)
    ebase = c * E + s * EPT

    for xw_h, out_h in ((xw_lo, out_lo), (xw_hi, out_hi)):
        def _zrow(i, carry):
            for k in range(H // 16):
                obuf[i, pl.ds(k * 16, 16)] = zero16
            return carry
        lax.fori_loop(0, RPT, _zrow, 0)
        pltpu.sync_copy(obuf, acc.at[pl.ds(s * RPT, RPT)])

        @pl.when(s == NS - 1)
        def _zero_rem():
            pltpu.sync_copy(obuf.at[pl.ds(0, REM)], acc.at[pl.ds(NS * RPT, REM)])
        plsc.subcore_barrier()

        def _chunk(j, carry):
            base = ebase + j * C
            pltpu.sync_copy(src_hbm.at[pl.ds(base, C)], sidx)
            pltpu.sync_copy(dst_hbm.at[pl.ds(base, C)], didx)
            pltpu.sync_copy(ew_hbm.at[pl.ds(base, C)], wv)
            pltpu.async_copy(xw_h.at[sidx], rows, sem).wait()

            def _edge16(g, cr):
                w16 = wv[pl.ds(g * 16, 16)]
                for e in range(16):
                    wspl = jnp.full((16,), w16[e], jnp.float32)
                    i = g * 16 + e
                    for k in range(H // 16):
                        sl = (i, pl.ds(k * 16, 16))
                        rows[sl] = rows[sl] * wspl
                return cr
            lax.fori_loop(0, C // 16, _edge16, 0)

            pltpu.sync_copy(rows, acc.at[didx], add=True)
            return carry
        lax.fori_loop(0, NCHUNK, _chunk, 0)
        plsc.subcore_barrier()

        pltpu.sync_copy(acc.at[pl.ds(s * RPT, RPT)], obuf)
        pltpu.sync_copy(obuf, out_h.at[pl.ds(c * N + s * RPT, RPT)])

        @pl.when(s == NS - 1)
        def _copy_rem():
            pltpu.sync_copy(acc.at[pl.ds(NS * RPT, REM)], obuf.at[pl.ds(0, REM)])
            pltpu.sync_copy(obuf.at[pl.ds(0, REM)],
                            out_h.at[pl.ds(c * N + NS * RPT, REM)])
        plsc.subcore_barrier()


_BM = 1000


def _mm_body(x_ref, w_ref, o_ref):
    o_ref[...] = jnp.dot(x_ref[...], w_ref[0], preferred_element_type=jnp.float32)


def _encode_mm(x_all, w_stacked):
    nblk = (2 * N) // _BM
    return pl.pallas_call(
        _mm_body,
        grid=(nblk,),
        in_specs=[
            pl.BlockSpec((_BM, D), lambda g: (g, 0)),
            pl.BlockSpec((1, D, D), lambda g: (g // (nblk // 2), 0, 0)),
        ],
        out_specs=pl.BlockSpec((_BM, D), lambda g: (g, 0)),
        out_shape=jax.ShapeDtypeStruct((2 * N, D), jnp.float32),
    )(x_all, w_stacked)


def _fuse_body(h1_ref, h2_ref, wom_ref, u_ref, wd1_ref, wd2_ref,
               emb_ref, al_ref, z1_ref, z2_ref):
    h1 = h1_ref[...]
    h2 = h2_ref[...]
    wom = wom_ref[...]
    u = u_ref[...]
    v1 = jnp.tanh(jnp.dot(h1, wom, preferred_element_type=jnp.float32))
    v2 = jnp.tanh(jnp.dot(h2, wom, preferred_element_type=jnp.float32))
    s1 = jnp.sum(v1 * u, axis=1, keepdims=True)
    s2 = jnp.sum(v2 * u, axis=1, keepdims=True)
    m = jnp.maximum(s1, s2)
    e1 = jnp.exp(s1 - m)
    e2 = jnp.exp(s2 - m)
    inv = 1.0 / (e1 + e2)
    a1 = e1 * inv
    a2 = e2 * inv
    emb = a1 * h1 + a2 * h2
    emb_ref[...] = emb
    al_ref[...] = jnp.concatenate([a1, a2], axis=1)
    z1_ref[...] = jnp.dot(emb, wd1_ref[...], preferred_element_type=jnp.float32)
    z2_ref[...] = jnp.dot(emb, wd2_ref[...], preferred_element_type=jnp.float32)


def _fuse(h1, h2, w_omega, u_row, W_dec1, W_dec2):
    nblk = N // _BM
    full = lambda g: (0, 0)
    return pl.pallas_call(
        _fuse_body,
        grid=(nblk,),
        in_specs=[
            pl.BlockSpec((_BM, D), lambda g: (g, 0)),
            pl.BlockSpec((_BM, D), lambda g: (g, 0)),
            pl.BlockSpec((D, D), full),
            pl.BlockSpec((1, D), full),
            pl.BlockSpec((D, D), full),
            pl.BlockSpec((D, D), full),
        ],
        out_specs=[
            pl.BlockSpec((_BM, D), lambda g: (g, 0)),
            pl.BlockSpec((_BM, 2), lambda g: (g, 0)),
            pl.BlockSpec((_BM, D), lambda g: (g, 0)),
            pl.BlockSpec((_BM, D), lambda g: (g, 0)),
        ],
        out_shape=[
            jax.ShapeDtypeStruct((N, D), jnp.float32),
            jax.ShapeDtypeStruct((N, 2), jnp.float32),
            jax.ShapeDtypeStruct((N, D), jnp.float32),
            jax.ShapeDtypeStruct((N, D), jnp.float32),
        ],
    )(h1, h2, w_omega, u_row, W_dec1, W_dec2)


def kernel(x1, x2, edge_index1, edge_weight1, edge_index2, edge_weight2,
           W_enc1, W_enc2, w_omega, u_omega, W_dec1, W_dec2):
    src_all = jnp.concatenate([edge_index1[0].astype(jnp.int32),
                               edge_index2[0].astype(jnp.int32) + N])
    dst_all = jnp.concatenate([edge_index1[1].astype(jnp.int32),
                               edge_index2[1].astype(jnp.int32)])
    ew_all = jnp.concatenate([edge_weight1, edge_weight2])

    x_all = jnp.concatenate([x1, x2], axis=0)
    w_enc = jnp.stack([W_enc1, W_enc2])
    xw_all = _encode_mm(x_all, w_enc)

    h_lo, h_hi = _spmm_pair(xw_all[:, :H], xw_all[:, H:], src_all, dst_all, ew_all)
    h_all = jnp.concatenate([h_lo, h_hi], axis=1)
    h1 = h_all[:N]
    h2 = h_all[N:]

    emb, alpha, z1, z2 = _fuse(h1, h2, w_omega, u_omega.reshape(1, D),
                               W_dec1, W_dec2)

    z_all = jnp.concatenate([z1, z2], axis=0)
    d_lo, d_hi = _spmm_pair(z_all[:, :H], z_all[:, H:], src_all, dst_all, ew_all)
    d_all = jnp.concatenate([d_lo, d_hi], axis=1)
    d1 = d_all[:N]
    d2 = d_all[N:]

    return (h1, h2, emb, alpha, d1, d2)


# speedup vs baseline: 6.5661x; 4.2892x over previous
import functools

import jax
import jax.numpy as jnp
from jax import lax
from jax.experimental import pallas as pl
from jax.experimental.pallas import tpu as pltpu
from jax.experimental.pallas import tpu_sc as plsc

N = 10000
E = 320000
D = 128

NC = 2
NS = 16
EPT = E // NS
C = 80
NCHUNK = EPT // C
RPT = 624
REM = N - NS * RPT
RC = 104


H = D // 2


NPAIR = NCHUNK // 2


@functools.partial(
    pl.kernel,
    out_type=[jax.ShapeDtypeStruct((2 * N, H), jnp.float32),
              jax.ShapeDtypeStruct((2 * N, H), jnp.float32)],
    mesh=plsc.VectorSubcoreMesh(core_axis_name="c", subcore_axis_name="s"),
    compiler_params=pltpu.CompilerParams(use_tc_tiling_on_sc=False),
    scratch_types=[
        pltpu.VMEM((EPT,), jnp.int32),
        pltpu.VMEM((NCHUNK, C), jnp.int32),
        pltpu.VMEM((EPT,), jnp.float32),
        pltpu.VMEM((C, H), jnp.float32),
        pltpu.VMEM((C, H), jnp.float32),
        pltpu.VMEM((RC, H), jnp.float32),
        pltpu.VMEM_SHARED((N, H), jnp.float32),
        pltpu.SemaphoreType.DMA,
        pltpu.SemaphoreType.DMA,
    ],
)
def _spmm_pair(xw_lo, xw_hi, src_hbm, dst2d_hbm, ew_hbm, out_lo, out_hi,
               sidx, didx, wv, rows0, rows1, obuf, acc, sem0, sem1):
    c = lax.axis_index("c")
    s = lax.axis_index("s")
    zero16 = jnp.zeros((16,), jnp.float32)
    ebase = c * E + s * EPT

    pltpu.sync_copy(src_hbm.at[pl.ds(ebase, EPT)], sidx)
    pltpu.sync_copy(ew_hbm.at[pl.ds(ebase, EPT)], wv)
    pltpu.sync_copy(dst2d_hbm.at[pl.ds((c * NS + s) * NCHUNK, NCHUNK)], didx)

    def _scale(rows, qC):
        for g in range(C // 16):
            w16 = wv[pl.ds(qC + g * 16, 16)]
            for e in range(16):
                wspl = jnp.full((16,), w16[e], jnp.float32)
                i = g * 16 + e
                for k in range(H // 16):
                    sl = (i, pl.ds(k * 16, 16))
                    rows[sl] = rows[sl] * wspl

    for xw_h, out_h in ((xw_lo, out_lo), (xw_hi, out_hi)):
        def _zrow(i, carry):
            for k in range(H // 16):
                obuf[i, pl.ds(k * 16, 16)] = zero16
            return carry
        lax.fori_loop(0, RC, _zrow, 0)
        for k in range(RPT // RC):
            pltpu.sync_copy(obuf, acc.at[pl.ds(s * RPT + k * RC, RC)])

        @pl.when(s == NS - 1)
        def _zero_rem():
            pltpu.sync_copy(obuf.at[pl.ds(0, REM)], acc.at[pl.ds(NS * RPT, REM)])
        plsc.subcore_barrier()

        pltpu.async_copy(xw_h.at[sidx.at[pl.ds(0, C)]], rows0, sem0)
        pltpu.async_copy(xw_h.at[sidx.at[pl.ds(C, C)]], rows1, sem1)

        def _pair(i, carry):
            q0 = 2 * i
            for p, (rows, sem) in enumerate(((rows0, sem0), (rows1, sem1))):
                q = q0 + p
                pltpu.make_async_copy(xw_h.at[sidx.at[pl.ds(0, C)]], rows, sem).wait()
                _scale(rows, q * C)
                pltpu.sync_copy(rows, acc.at[didx.at[q]], add=True)

                @pl.when(i < NPAIR - 1)
                def _next():
                    pltpu.async_copy(
                        xw_h.at[sidx.at[pl.ds((q + 2) * C, C)]], rows, sem)
            return carry
        lax.fori_loop(0, NPAIR, _pair, 0)
        plsc.subcore_barrier()

        for k in range(RPT // RC):
            r = s * RPT + k * RC
            pltpu.sync_copy(acc.at[pl.ds(r, RC)], obuf)
            pltpu.sync_copy(obuf, out_h.at[pl.ds(c * N + r, RC)])

        @pl.when(s == NS - 1)
        def _copy_rem():
            pltpu.sync_copy(acc.at[pl.ds(NS * RPT, REM)], obuf.at[pl.ds(0, REM)])
            pltpu.sync_copy(obuf.at[pl.ds(0, REM)],
                            out_h.at[pl.ds(c * N + NS * RPT, REM)])
        plsc.subcore_barrier()


_BM = 1000


def _mm_body(x_ref, w_ref, o_ref):
    o_ref[...] = jnp.dot(x_ref[...], w_ref[0], preferred_element_type=jnp.float32)


def _encode_mm(x_all, w_stacked):
    nblk = (2 * N) // _BM
    return pl.pallas_call(
        _mm_body,
        grid=(nblk,),
        in_specs=[
            pl.BlockSpec((_BM, D), lambda g: (g, 0)),
            pl.BlockSpec((1, D, D), lambda g: (g // (nblk // 2), 0, 0)),
        ],
        out_specs=pl.BlockSpec((_BM, D), lambda g: (g, 0)),
        out_shape=jax.ShapeDtypeStruct((2 * N, D), jnp.float32),
    )(x_all, w_stacked)


def _fuse_body(h1_ref, h2_ref, wom_ref, u_ref, wd1_ref, wd2_ref,
               emb_ref, al_ref, z1_ref, z2_ref):
    h1 = h1_ref[...]
    h2 = h2_ref[...]
    wom = wom_ref[...]
    u = u_ref[...]
    v1 = jnp.tanh(jnp.dot(h1, wom, preferred_element_type=jnp.float32))
    v2 = jnp.tanh(jnp.dot(h2, wom, preferred_element_type=jnp.float32))
    s1 = jnp.sum(v1 * u, axis=1, keepdims=True)
    s2 = jnp.sum(v2 * u, axis=1, keepdims=True)
    m = jnp.maximum(s1, s2)
    e1 = jnp.exp(s1 - m)
    e2 = jnp.exp(s2 - m)
    inv = 1.0 / (e1 + e2)
    a1 = e1 * inv
    a2 = e2 * inv
    emb = a1 * h1 + a2 * h2
    emb_ref[...] = emb
    al_ref[...] = jnp.concatenate([a1, a2], axis=1)
    z1_ref[...] = jnp.dot(emb, wd1_ref[...], preferred_element_type=jnp.float32)
    z2_ref[...] = jnp.dot(emb, wd2_ref[...], preferred_element_type=jnp.float32)


def _fuse(h1, h2, w_omega, u_row, W_dec1, W_dec2):
    nblk = N // _BM
    full = lambda g: (0, 0)
    return pl.pallas_call(
        _fuse_body,
        grid=(nblk,),
        in_specs=[
            pl.BlockSpec((_BM, D), lambda g: (g, 0)),
            pl.BlockSpec((_BM, D), lambda g: (g, 0)),
            pl.BlockSpec((D, D), full),
            pl.BlockSpec((1, D), full),
            pl.BlockSpec((D, D), full),
            pl.BlockSpec((D, D), full),
        ],
        out_specs=[
            pl.BlockSpec((_BM, D), lambda g: (g, 0)),
            pl.BlockSpec((_BM, 2), lambda g: (g, 0)),
            pl.BlockSpec((_BM, D), lambda g: (g, 0)),
            pl.BlockSpec((_BM, D), lambda g: (g, 0)),
        ],
        out_shape=[
            jax.ShapeDtypeStruct((N, D), jnp.float32),
            jax.ShapeDtypeStruct((N, 2), jnp.float32),
            jax.ShapeDtypeStruct((N, D), jnp.float32),
            jax.ShapeDtypeStruct((N, D), jnp.float32),
        ],
    )(h1, h2, w_omega, u_row, W_dec1, W_dec2)


def kernel(x1, x2, edge_index1, edge_weight1, edge_index2, edge_weight2,
           W_enc1, W_enc2, w_omega, u_omega, W_dec1, W_dec2):
    src_all = jnp.concatenate([edge_index1[0].astype(jnp.int32),
                               edge_index2[0].astype(jnp.int32) + N])
    dst_all = jnp.concatenate([edge_index1[1].astype(jnp.int32),
                               edge_index2[1].astype(jnp.int32)])
    ew_all = jnp.concatenate([edge_weight1, edge_weight2])

    x_all = jnp.concatenate([x1, x2], axis=0)
    w_enc = jnp.stack([W_enc1, W_enc2])
    xw_all = _encode_mm(x_all, w_enc)

    dst2d = dst_all.reshape(2 * NS * NCHUNK, C)

    h_lo, h_hi = _spmm_pair(xw_all[:, :H], xw_all[:, H:], src_all, dst2d, ew_all)
    h_all = jnp.concatenate([h_lo, h_hi], axis=1)
    h1 = h_all[:N]
    h2 = h_all[N:]

    emb, alpha, z1, z2 = _fuse(h1, h2, w_omega, u_omega.reshape(1, D),
                               W_dec1, W_dec2)

    z_all = jnp.concatenate([z1, z2], axis=0)
    d_lo, d_hi = _spmm_pair(z_all[:, :H], z_all[:, H:], src_all, dst2d, ew_all)
    d_all = jnp.concatenate([d_lo, d_hi], axis=1)
    d1 = d_all[:N]
    d2 = d_all[N:]

    return (h1, h2, emb, alpha, d1, d2)
